# baseline (speedup 1.0000x reference)
import jax
import jax.numpy as jnp
from jax import lax
from jax.experimental import pallas as pl
from jax.experimental.pallas import tpu as pltpu
from jax.experimental.pallas import tpu_sc as plsc
from jax._src.pallas import mpmd

_R = 256
_C = 224
_K = 50
_NC = 2
_NS = 16
_NW = _NC * _NS
_RPW = _R // _NW
_L = 16
_XW = _RPW * _C
_OW = _RPW * _K
_NCHUNK = _OW // _L
_HX = _NS * _XW
_HA = _NS * _OW
_MAGIC = 0x5CA11ED
_POLL_MAX = 1 << 20


def _scs_body(x_hbm, a_hbm, mg_hbm, out_hbm, x_sh, a_sh, f_sh, sem1, sem2):
    cid = lax.axis_index("c")
    cp1 = pltpu.async_copy(x_hbm.at[pl.ds(cid * _HX, _HX)], x_sh, sem1)
    cp2 = pltpu.async_copy(a_hbm.at[pl.ds(cid * _HA, _HA)], a_sh, sem2)
    cp1.wait()
    cp2.wait()
    pltpu.sync_copy(mg_hbm, f_sh)


def _tec_body(x_hbm, a_hbm, mg_hbm, out_hbm, x_sh, a_sh, f_sh, sem1, sem2):
    cid = lax.axis_index("c")
    sid = lax.axis_index("s")
    wid = cid * _NS + sid
    xbase = wid * _XW
    obase = wid * _OW

    def inner(x_v, a_v, o_v, f_v, sem_x, sem_a):
        my_flag = f_sh.at[pl.ds(sid * _L, _L)]

        def poll(max_iters):
            def cond(carry):
                i, ok = carry
                return jnp.logical_and(i < max_iters, ok == 0)

            def body(carry):
                i, ok = carry
                pltpu.sync_copy(my_flag, f_v)
                seen = jnp.all(f_v[...] == _MAGIC).astype(jnp.int32)
                return (i + 1, seen)

            _, ok = lax.while_loop(cond, body, (jnp.int32(0), jnp.int32(0)))
            return ok

        ok = poll(64)

        @pl.when(ok == 1)
        def _():
            cp_x = pltpu.async_copy(x_sh.at[pl.ds(sid * _XW, _XW)], x_v, sem_x)
            cp_a = pltpu.async_copy(a_sh.at[pl.ds(sid * _OW, _OW)], a_v, sem_a)
            cp_x.wait()
            cp_a.wait()

        @pl.when(ok == 0)
        def _():
            cp_x = pltpu.async_copy(x_hbm.at[pl.ds(xbase, _XW)], x_v, sem_x)
            cp_a = pltpu.async_copy(a_hbm.at[pl.ds(obase, _OW)], a_v, sem_a)
            cp_x.wait()
            cp_a.wait()

        lanes = lax.iota(jnp.int32, _L)
        for t in range(_NCHUNK):
            idx = a_v[pl.ds(t * _L, _L)]
            g = ((lanes + t * _L) // _K) * _C + idx
            o_v[pl.ds(t * _L, _L)] = plsc.load_gather(x_v, [g])
        pltpu.sync_copy(o_v, out_hbm.at[pl.ds(obase, _OW)])

        @pl.when(ok == 0)
        def _():
            poll(_POLL_MAX)

        f_v[...] = jnp.zeros((_L,), jnp.int32)
        pltpu.sync_copy(f_v, my_flag)

    pl.run_scoped(
        inner,
        pltpu.VMEM((_XW,), jnp.float32),
        pltpu.VMEM((_OW,), jnp.int32),
        pltpu.VMEM((_OW,), jnp.float32),
        pltpu.VMEM((_L,), jnp.int32),
        pltpu.SemaphoreType.DMA,
        pltpu.SemaphoreType.DMA,
    )


@jax.jit
def _gather(xf, af):
    smesh = plsc.ScalarSubcoreMesh(axis_name="c", num_cores=_NC)
    vmesh = plsc.VectorSubcoreMesh(
        core_axis_name="c", subcore_axis_name="s",
        num_cores=_NC, num_subcores=_NS,
    )
    mg = jnp.full((_NS * _L,), _MAGIC, dtype=jnp.int32)
    (out,) = mpmd.mpmd_map(
        [(smesh, _scs_body), (vmesh, _tec_body)],
        out_types=[jax.ShapeDtypeStruct((_R * _K,), jnp.float32)],
        scratch_types=[
            pltpu.VMEM_SHARED((_HX,), jnp.float32),
            pltpu.VMEM_SHARED((_HA,), jnp.int32),
            pltpu.VMEM_SHARED((_NS * _L,), jnp.int32),
            pltpu.SemaphoreType.DMA @ smesh,
            pltpu.SemaphoreType.DMA @ smesh,
        ],
        compiler_params=pltpu.CompilerParams(needs_layout_passes=False),
    )(xf, af, mg)
    return out


def kernel(x, a):
    xf = x.reshape(_R * _C)
    af = a.reshape(_R * _K)
    out = _gather(xf, af)
    return out.reshape(1, _R, _K)


# speedup vs baseline: 1.0187x; 1.0187x over previous
import jax
import jax.numpy as jnp
from jax import lax
from jax.experimental import pallas as pl
from jax.experimental.pallas import tpu as pltpu
from jax.experimental.pallas import tpu_sc as plsc

_R = 256
_C = 224
_K = 50
_NC = 2
_NS = 16
_NW = _NC * _NS
_RPW = _R // _NW
_L = 16
_XW = _RPW * _C
_OW = _RPW * _K
_NCHUNK = _OW // _L

def _body(x_hbm, a_hbm, out_hbm, x_v, a_v, o_v, sem_x, sem_a):
    wid = lax.axis_index("c") * _NS + lax.axis_index("s")
    xbase = wid * _XW
    obase = wid * _OW
    half = _XW // 2
    cp_a = pltpu.async_copy(a_hbm.at[pl.ds(obase, _OW)], a_v, sem_a)
    cp_x1 = pltpu.async_copy(
        x_hbm.at[pl.ds(xbase, half)], x_v.at[pl.ds(0, half)], sem_x)
    cp_x2 = pltpu.async_copy(
        x_hbm.at[pl.ds(xbase + half, half)], x_v.at[pl.ds(half, half)], sem_x)
    lanes = lax.iota(jnp.int32, _L)

    def gather_chunk(t):
        idx = a_v[pl.ds(t * _L, _L)]
        g = ((lanes + t * _L) // _K) * _C + idx
        o_v[pl.ds(t * _L, _L)] = plsc.load_gather(x_v, [g])

    cp_a.wait()
    cp_x1.wait()
    for t in range(12):
        gather_chunk(t)
    cp_x2.wait()
    for t in range(12, _NCHUNK):
        gather_chunk(t)
    pltpu.sync_copy(o_v, out_hbm.at[pl.ds(obase, _OW)])


@jax.jit
def _gather(xf, af):
    mesh = plsc.VectorSubcoreMesh(
        core_axis_name="c", subcore_axis_name="s",
        num_cores=_NC, num_subcores=_NS,
    )
    return pl.kernel(
        _body,
        out_type=jax.ShapeDtypeStruct((_R * _K,), jnp.float32),
        mesh=mesh,
        scratch_types=[
            pltpu.VMEM((_XW,), jnp.float32),
            pltpu.VMEM((_OW,), jnp.int32),
            pltpu.VMEM((_OW,), jnp.float32),
            pltpu.SemaphoreType.DMA,
            pltpu.SemaphoreType.DMA,
        ],
        compiler_params=pltpu.CompilerParams(needs_layout_passes=False),
    )(xf, af)


def kernel(x, a):
    xf = x.reshape(_R * _C)
    af = a.reshape(_R * _K)
    out = _gather(xf, af)
    return out.reshape(1, _R, _K)


# speedup vs baseline: 1.0201x; 1.0014x over previous
import jax
import jax.numpy as jnp
from jax import lax
from jax.experimental import pallas as pl
from jax.experimental.pallas import tpu as pltpu
from jax.experimental.pallas import tpu_sc as plsc

_R = 256
_C = 224
_K = 50
_NC = 2
_NS = 16
_NW = _NC * _NS
_RPW = _R // _NW
_L = 16
_XW = _RPW * _C
_OW = _RPW * _K
_NCHUNK = _OW // _L

def _body(x_hbm, a_hbm, out_hbm, x_v, a_v, o_v, sem_x, sem_a):
    wid = lax.axis_index("c") * _NS + lax.axis_index("s")
    xbase = wid * _XW
    obase = wid * _OW
    half = _XW // 2
    cp_a = pltpu.async_copy(a_hbm.at[pl.ds(obase, _OW)], a_v, sem_a)
    cp_x1 = pltpu.async_copy(
        x_hbm.at[pl.ds(xbase, half)], x_v.at[pl.ds(0, half)], sem_x)
    cp_x2 = pltpu.async_copy(
        x_hbm.at[pl.ds(xbase + half, half)], x_v.at[pl.ds(half, half)], sem_x)
    lanes = lax.iota(jnp.int32, _L)

    cp_a.wait()
    cp_x1.wait()

    @plsc.parallel_loop(0, 12 * _L, step=_L, unroll=4)
    def _(p):
        idx = a_v[pl.ds(p, _L)]
        g = ((lanes + p) // _K) * _C + idx
        o_v[pl.ds(p, _L)] = plsc.load_gather(x_v, [g])

    cp_x2.wait()

    @plsc.parallel_loop(12 * _L, _NCHUNK * _L, step=_L, unroll=4)
    def _(p):
        idx = a_v[pl.ds(p, _L)]
        g = ((lanes + p) // _K) * _C + idx
        o_v[pl.ds(p, _L)] = plsc.load_gather(x_v, [g])
    pltpu.sync_copy(o_v, out_hbm.at[pl.ds(obase, _OW)])


@jax.jit
def _gather(xf, af):
    mesh = plsc.VectorSubcoreMesh(
        core_axis_name="c", subcore_axis_name="s",
        num_cores=_NC, num_subcores=_NS,
    )
    return pl.kernel(
        _body,
        out_type=jax.ShapeDtypeStruct((_R * _K,), jnp.float32),
        mesh=mesh,
        scratch_types=[
            pltpu.VMEM((_XW,), jnp.float32),
            pltpu.VMEM((_OW,), jnp.int32),
            pltpu.VMEM((_OW,), jnp.float32),
            pltpu.SemaphoreType.DMA,
            pltpu.SemaphoreType.DMA,
        ],
        compiler_params=pltpu.CompilerParams(needs_layout_passes=False),
    )(xf, af)


def kernel(x, a):
    xf = x.reshape(_R * _C)
    af = a.reshape(_R * _K)
    out = _gather(xf, af)
    return out.reshape(1, _R, _K)
